# baseline (speedup 1.0000x reference)
import functools

import jax
import jax.numpy as jnp
from jax import lax
from jax.experimental import pallas as pl
from jax.experimental.pallas import tpu as pltpu
from jax.experimental.pallas import tpu_sc as plsc

_B = 64
_R = 384
_W = 128
_N = _R * _W
_NC = 2
_NS = 16
_NW = _NC * _NS
_RPW = _R // _NW
_WIN = 8
_CHUNK = _RPW * _W
_ILV = 4


def _sc_mode_body(x_hbm, out_hbm, x_v, mode_v, addr_v, h0, h1, h2, h3, sem):
    wid = lax.axis_index("s") * _NC + lax.axis_index("c")
    r0 = wid * _RPW
    astart = pl.multiple_of((r0 // 8) * 8, 8)

    lane = lax.iota(jnp.int32, 16)
    zeros16 = jnp.zeros((16,), jnp.int32)
    hists = (h0, h1, h2, h3)

    def zero_body(j, c):
        for h in hists:
            h[pl.ds(j * 16, 16)] = zeros16
        return c

    lax.fori_loop(0, 256, zero_body, 0)

    for rnd in range(2):
        wa = astart + rnd * _WIN
        copies = [
            pltpu.make_async_copy(
                x_hbm.at[b, pl.ds(wa, _WIN), :],
                x_v.at[pl.ds(b * _WIN, _WIN), :], sem)
            for b in range(_B)
        ]
        for c in copies:
            c.start()
        for c in copies:
            c.wait()

        lo = jnp.maximum(r0, wa) - wa
        hi = jnp.minimum(r0 + _RPW, wa + _WIN) - wa

        def iter_body(gidx, c, rnd=rnd):
            row = jnp.right_shift(gidx, 1)
            l0 = (gidx & 1) * 64
            gbase = (rnd * 16 + gidx) * 256
            for b in range(_B):
                vis = [x_v[b * _WIN + row, pl.ds(l0 + 16 * k, 16)
                           ].astype(jnp.int32)
                       for k in range(_ILV)]
                for k in range(_ILV):
                    addr_v[pl.ds((b * _ILV + k) * 16, 16)] = \
                        vis[k] * 16 + lane
            bests = [zeros16] * _ILV
            for b in range(_B):
                addrs = [addr_v[pl.ds((b * _ILV + k) * 16, 16)]
                         for k in range(_ILV)]
                ents = [jnp.maximum(plsc.load_gather(hists[k], [addrs[k]]),
                                    gbase) + 1
                        for k in range(_ILV)]
                for k in range(_ILV):
                    plsc.store_scatter(hists[k], [addrs[k]], ents[k])
                for k in range(_ILV):
                    key = jnp.left_shift(ents[k] - gbase, 12) + (
                        4095 - addrs[k])
                    bests[k] = jnp.maximum(bests[k], key)
            moff = (wa + row - r0) * 128 + l0
            for k in range(_ILV):
                mode_v[pl.ds(moff + 16 * k, 16)] = \
                    jnp.right_shift(4095 - (bests[k] & 4095),
                                    4).astype(jnp.float32)
            return c

        lax.fori_loop(lo * 2, hi * 2, iter_body, 0)

    pltpu.sync_copy(mode_v, out_hbm.at[pl.ds(wid * _CHUNK, _CHUNK)])


_sc_mode = functools.partial(
    pl.kernel,
    out_type=jax.ShapeDtypeStruct((_N,), jnp.float32),
    mesh=plsc.VectorSubcoreMesh(core_axis_name="c", subcore_axis_name="s"),
    scratch_types=[
        pltpu.VMEM((_B * _WIN, _W), jnp.float32),
        pltpu.VMEM((_CHUNK,), jnp.float32),
        pltpu.VMEM((_B * _ILV * 16,), jnp.int32),
        pltpu.VMEM((4096,), jnp.int32),
        pltpu.VMEM((4096,), jnp.int32),
        pltpu.VMEM((4096,), jnp.int32),
        pltpu.VMEM((4096,), jnp.int32),
        pltpu.SemaphoreType.DMA,
    ],
    compiler_params=pltpu.CompilerParams(needs_layout_passes=False),
)(_sc_mode_body)


def _expand_body(x_ref, m_ref, bg_ref, fg_ref):
    x = x_ref[...]
    bg = jnp.broadcast_to(m_ref[...][None], x.shape)
    bg_ref[...] = bg
    fg_ref[...] = jnp.abs(x - bg)


def kernel(input):
    B, C, H, W = input.shape
    x3 = input.reshape(B, C * H, W)
    mode = _sc_mode(x3).reshape(_R, _W)
    RB = 48
    bg, fg = pl.pallas_call(
        _expand_body,
        grid=(_R // RB,),
        in_specs=[pl.BlockSpec((B, RB, W), lambda i: (0, i, 0)),
                  pl.BlockSpec((RB, W), lambda i: (i, 0))],
        out_specs=[pl.BlockSpec((B, RB, W), lambda i: (0, i, 0)),
                   pl.BlockSpec((B, RB, W), lambda i: (0, i, 0))],
        out_shape=[jax.ShapeDtypeStruct((B, C * H, W), jnp.float32),
                   jax.ShapeDtypeStruct((B, C * H, W), jnp.float32)],
    )(x3, mode)
    return bg.reshape(input.shape), fg.reshape(input.shape)
